# Initial kernel scaffold; baseline (speedup 1.0000x reference)
import jax
import jax.numpy as jnp
from jax.experimental import pallas as pl


def kernel(x, edge_index, edge_type, W_conv1, Wa1, Wg1, Wb1, Wa2, Wg2, Wb2, Wp1, bp1, Wp2, bp2, Wr1, br1, Wr2, br2):
    raise NotImplementedError("write your pallas kernel here")


# speedup vs baseline: 1.8167x; 1.8167x over previous
import functools

import jax
import jax.numpy as jnp
from jax import lax
from jax.experimental import pallas as pl
from jax.experimental.pallas import tpu as pltpu

N = 100000
E = 1600000
T = 7
C_IN = 4
C = 32
CB = 8
H = 32

BN = 2000
GRID = N // BN


def _tc1_body(acc1_ref, wconv_ref, wa1_ref, wg1_ref, h1_ref, ys2_ref):
    a = acc1_ref[...]
    w = wconv_ref[...]
    s = jnp.zeros((BN, C), dtype=jnp.float32)
    for t in range(T):
        s = s + jnp.dot(a[t], w[t], preferred_element_type=jnp.float32)
    h1 = jax.nn.relu(s * (1.0 / 7.0))
    h1_ref[...] = h1
    z2 = jax.nn.relu(jnp.dot(h1, wa1_ref[...], preferred_element_type=jnp.float32))
    wg = wg1_ref[...]
    for t in range(T):
        ys2_ref[t] = jnp.dot(z2, wg[t], preferred_element_type=jnp.float32)


def _tc1(acc1, wconv, wa1, wg1):
    return pl.pallas_call(
        _tc1_body,
        grid=(GRID,),
        in_specs=[
            pl.BlockSpec((T, BN, C_IN), lambda i: (0, i, 0)),
            pl.BlockSpec((T, C_IN, C), lambda i: (0, 0, 0)),
            pl.BlockSpec((C, CB), lambda i: (0, 0)),
            pl.BlockSpec((T, CB, CB), lambda i: (0, 0, 0)),
        ],
        out_specs=[
            pl.BlockSpec((BN, C), lambda i: (i, 0)),
            pl.BlockSpec((T, BN, CB), lambda i: (0, i, 0)),
        ],
        out_shape=[
            jax.ShapeDtypeStruct((N, C), jnp.float32),
            jax.ShapeDtypeStruct((T, N, CB), jnp.float32),
        ],
    )(acc1, wconv, wa1, wg1)


def _tc2_body(acc_ref, h1_ref, wb_ref, wa_ref, wg_ref, h2_ref, ys3_ref):
    a = (acc_ref[0] + acc_ref[1]) * (1.0 / 7.0)
    z = jax.nn.relu(a)
    z = jnp.dot(z, wb_ref[...], preferred_element_type=jnp.float32)
    h2 = jax.nn.relu(h1_ref[...] + z)
    h2_ref[...] = h2
    z3 = jax.nn.relu(jnp.dot(h2, wa_ref[...], preferred_element_type=jnp.float32))
    wg = wg_ref[...]
    for t in range(T):
        ys3_ref[t] = jnp.dot(z3, wg[t], preferred_element_type=jnp.float32)


def _tc2(acc2, h1, wb1, wa2, wg2):
    return pl.pallas_call(
        _tc2_body,
        grid=(GRID,),
        in_specs=[
            pl.BlockSpec((2, BN, CB), lambda i: (0, i, 0)),
            pl.BlockSpec((BN, C), lambda i: (i, 0)),
            pl.BlockSpec((CB, C), lambda i: (0, 0)),
            pl.BlockSpec((C, CB), lambda i: (0, 0)),
            pl.BlockSpec((T, CB, CB), lambda i: (0, 0, 0)),
        ],
        out_specs=[
            pl.BlockSpec((BN, C), lambda i: (i, 0)),
            pl.BlockSpec((T, BN, CB), lambda i: (0, i, 0)),
        ],
        out_shape=[
            jax.ShapeDtypeStruct((N, C), jnp.float32),
            jax.ShapeDtypeStruct((T, N, CB), jnp.float32),
        ],
    )(acc2, h1, wb1, wa2, wg2)


def _tc3_body(acc_ref, h2_ref, wb_ref, wp1_ref, bp1_ref, wp2_ref, bp2_ref,
              wr1_ref, br1_ref, wr2_ref, br2_ref, logits_ref, reg_ref):
    a = (acc_ref[0] + acc_ref[1]) * (1.0 / 7.0)
    z = jax.nn.relu(a)
    z = jnp.dot(z, wb_ref[...], preferred_element_type=jnp.float32)
    h3 = jax.nn.relu(h2_ref[...] + z)
    p = jax.nn.relu(jnp.dot(h3, wp1_ref[...], preferred_element_type=jnp.float32)
                    + bp1_ref[...])
    logits_ref[...] = (jnp.dot(p, wp2_ref[...], preferred_element_type=jnp.float32)
                       + bp2_ref[...])
    r = jax.nn.relu(jnp.dot(h3, wr1_ref[...], preferred_element_type=jnp.float32)
                    + br1_ref[...])
    reg_ref[...] = (jnp.dot(r, wr2_ref[...], preferred_element_type=jnp.float32)
                    + br2_ref[...])


def _tc3(acc3, h2, wb2, wp1, bp1, wp2, bp2, wr1, br1, wr2, br2):
    full = lambda *s: pl.BlockSpec(s, lambda i: tuple(0 for _ in s))
    return pl.pallas_call(
        _tc3_body,
        grid=(GRID,),
        in_specs=[
            pl.BlockSpec((2, BN, CB), lambda i: (0, i, 0)),
            pl.BlockSpec((BN, C), lambda i: (i, 0)),
            full(CB, C), full(C, H), full(1, H), full(H, 2), full(1, 2),
            full(C, H), full(1, H), full(H, 4), full(1, 4),
        ],
        out_specs=[
            pl.BlockSpec((BN, 2), lambda i: (i, 0)),
            pl.BlockSpec((BN, 4), lambda i: (i, 0)),
        ],
        out_shape=[
            jax.ShapeDtypeStruct((N, 2), jnp.float32),
            jax.ShapeDtypeStruct((N, 4), jnp.float32),
        ],
    )(acc3, h2, wb2, wp1, bp1.reshape(1, H), wp2, bp2.reshape(1, 2),
      wr1, br1.reshape(1, H), wr2, br2.reshape(1, 4))


def kernel(x, edge_index, edge_type, W_conv1, Wa1, Wg1, Wb1, Wa2, Wg2, Wb2,
           Wp1, bp1, Wp2, bp2, Wr1, br1, Wr2, br2):
    src = edge_index[0]
    dst = edge_index[1]

    acc1 = jax.ops.segment_sum(x[src], edge_type * N + dst, num_segments=T * N)
    acc1 = acc1.reshape(T, N, C_IN)
    h1, ys2 = _tc1(acc1, W_conv1, Wa1, Wg1)

    g = edge_type * N + src
    acc2 = jax.ops.segment_sum(ys2.reshape(T * N, CB)[g], dst, num_segments=N)
    acc2 = jnp.stack([acc2, jnp.zeros_like(acc2)])
    h2, ys3 = _tc2(acc2, h1, Wb1, Wa2, Wg2)

    acc3 = jax.ops.segment_sum(ys3.reshape(T * N, CB)[g], dst, num_segments=N)
    acc3 = jnp.stack([acc3, jnp.zeros_like(acc3)])
    logits, reg = _tc3(acc3, h2, Wb2, Wp1, bp1, Wp2, bp2, Wr1, br1, Wr2, br2)
    return (logits, reg)


# speedup vs baseline: 21.9517x; 12.0834x over previous
import functools

import jax
import jax.numpy as jnp
from jax import lax
from jax.experimental import pallas as pl
from jax.experimental.pallas import tpu as pltpu
from jax.experimental.pallas import tpu_sc as plsc

N = 100000
E = 1600000
T = 7
C_IN = 4
C = 32
CB = 8
H = 32

BN = 2000
GRID = N // BN

EP = 1605632
EROWS = EP // 128
B_E = 1024
NB23 = 49
NB1 = 98
TRASH1 = 2 * N
ACC1_ROWS = 2 * N + 64
ACC23_ROWS = N + 96
_SC_MESH = plsc.VectorSubcoreMesh(core_axis_name="c", subcore_axis_name="s")


def _prep_body(s_ref, d_ref, t_ref, g_ref, gx_ref, s1a_ref, s1b_ref):
    s = s_ref[...]
    d = d_ref[...]
    t = t_ref[...]
    g_ref[...] = t * N + s
    gx_ref[...] = 2 * s + (t & 1)
    valid = d < N
    tp = t >> 1
    s1a_ref[...] = jnp.where((t < 4) & valid, tp * N + d, TRASH1)
    s1b_ref[...] = jnp.where((t >= 4) & valid, (tp - 2) * N + d, TRASH1)


def _prep(srcp, dstp, typep):
    spec = pl.BlockSpec((64, 128), lambda i: (i, 0))
    return pl.pallas_call(
        _prep_body,
        grid=(EROWS // 64,),
        in_specs=[spec, spec, spec],
        out_specs=[spec, spec, spec, spec],
        out_shape=[jax.ShapeDtypeStruct((EROWS, 128), jnp.int32)] * 4,
    )(srcp, dstp, typep)


def _sc_conv1_body(xd_hbm, gx2_hbm, s1ab_hbm, zeros_hbm, out_hbm,
                   acc, gbuf, sbuf, rows, gsem, ssem):
    c = lax.axis_index("c")
    s = lax.axis_index("s")
    pltpu.sync_copy(zeros_hbm, acc.at[pl.ds(s * (ACC1_ROWS // 16), ACC1_ROWS // 16)])
    plsc.subcore_barrier()
    base128 = s * (NB1 * 8)

    def blk(b):
        ro = base128 + b * 8
        pltpu.sync_copy(gx2_hbm.at[pl.ds(ro, 8)], gbuf)
        pltpu.sync_copy(s1ab_hbm.at[pl.ds(c * EROWS + ro, 8)], sbuf)
        descs = [pltpu.async_copy(xd_hbm.at[gbuf.at[j]],
                                  rows.at[pl.ds(j * 128, 128)], gsem)
                 for j in range(8)]
        for dsc in descs:
            dsc.wait()
        descs = [pltpu.async_copy(rows.at[pl.ds(j * 128, 128)],
                                  acc.at[sbuf.at[j]], ssem, add=True)
                 for j in range(8)]
        for dsc in descs:
            dsc.wait()

    pl.loop(0, NB1)(blk)
    plsc.subcore_barrier()

    @pl.when(s < 15)
    def _():
        pltpu.sync_copy(acc.at[pl.ds(s * 12504, 12504)],
                        out_hbm.at[pl.ds(c * 2 * N + s * 12504, 12504)])

    @pl.when(s == 15)
    def _():
        pltpu.sync_copy(acc.at[pl.ds(15 * 12504, 12440)],
                        out_hbm.at[pl.ds(c * 2 * N + 15 * 12504, 12440)])


_sc_conv1 = functools.partial(
    pl.kernel,
    out_type=jax.ShapeDtypeStruct((4 * N, 2 * C_IN), jnp.float32),
    mesh=_SC_MESH,
    compiler_params=pltpu.CompilerParams(use_tc_tiling_on_sc=False),
    scratch_types=[
        pltpu.VMEM_SHARED((ACC1_ROWS, 2 * C_IN), jnp.float32),
        pltpu.VMEM((8, 128), jnp.int32),
        pltpu.VMEM((8, 128), jnp.int32),
        pltpu.VMEM((B_E, 2 * C_IN), jnp.float32),
        pltpu.SemaphoreType.DMA,
        pltpu.SemaphoreType.DMA,
    ],
)(_sc_conv1_body)


def _sc_gconv_body(tab_hbm, g2_hbm, d2_hbm, zeros_hbm, out_hbm,
                   acc, gbuf, sbuf, rows, gsem, ssem):
    c = lax.axis_index("c")
    s = lax.axis_index("s")
    pltpu.sync_copy(zeros_hbm, acc.at[pl.ds(s * (ACC23_ROWS // 16), ACC23_ROWS // 16)])
    plsc.subcore_barrier()
    base128 = (c * 16 + s) * (NB23 * 8)

    def blk(b):
        ro = base128 + b * 8
        pltpu.sync_copy(g2_hbm.at[pl.ds(ro, 8)], gbuf)
        pltpu.sync_copy(d2_hbm.at[pl.ds(ro, 8)], sbuf)
        descs = [pltpu.async_copy(tab_hbm.at[gbuf.at[j]],
                                  rows.at[pl.ds(j * 128, 128)], gsem)
                 for j in range(8)]
        for dsc in descs:
            dsc.wait()
        descs = [pltpu.async_copy(rows.at[pl.ds(j * 128, 128)],
                                  acc.at[sbuf.at[j]], ssem, add=True)
                 for j in range(8)]
        for dsc in descs:
            dsc.wait()

    pl.loop(0, NB23)(blk)
    plsc.subcore_barrier()
    @pl.when(s < 15)
    def _():
        pltpu.sync_copy(acc.at[pl.ds(s * 6256, 6256)],
                        out_hbm.at[pl.ds(c * N + s * 6256, 6256)])

    @pl.when(s == 15)
    def _():
        pltpu.sync_copy(acc.at[pl.ds(15 * 6256, 6160)],
                        out_hbm.at[pl.ds(c * N + 15 * 6256, 6160)])


_sc_gconv = functools.partial(
    pl.kernel,
    out_type=jax.ShapeDtypeStruct((2 * N, CB), jnp.float32),
    mesh=_SC_MESH,
    compiler_params=pltpu.CompilerParams(use_tc_tiling_on_sc=False),
    scratch_types=[
        pltpu.VMEM_SHARED((ACC23_ROWS, CB), jnp.float32),
        pltpu.VMEM((8, 128), jnp.int32),
        pltpu.VMEM((8, 128), jnp.int32),
        pltpu.VMEM((B_E, CB), jnp.float32),
        pltpu.SemaphoreType.DMA,
        pltpu.SemaphoreType.DMA,
    ],
)(_sc_gconv_body)


def _tc1_body(acc1_ref, wconv_ref, wa1_ref, wg1_ref, h1_ref, ys2_ref):
    a = acc1_ref[...]
    w = wconv_ref[...]
    s = jnp.zeros((BN, C), dtype=jnp.float32)
    for t in range(T):
        a_t = a[t >> 1][:, (t & 1) * 4:(t & 1) * 4 + 4]
        s = s + jnp.dot(a_t, w[t], preferred_element_type=jnp.float32)
    h1 = jax.nn.relu(s * (1.0 / 7.0))
    h1_ref[...] = h1
    z2 = jax.nn.relu(jnp.dot(h1, wa1_ref[...], preferred_element_type=jnp.float32))
    wg = wg1_ref[...]
    for t in range(T):
        ys2_ref[t] = jnp.dot(z2, wg[t], preferred_element_type=jnp.float32)


def _mkxd_body(x_ref, xd_ref):
    xr = x_ref[...]
    z = jnp.zeros((BN, C_IN), jnp.float32)
    a = jnp.concatenate([xr, z], axis=1)
    b = jnp.concatenate([z, xr], axis=1)
    xd_ref[...] = jnp.stack([a, b], axis=1).reshape(2 * BN, 2 * C_IN)


def _mkxd(x):
    return pl.pallas_call(
        _mkxd_body,
        grid=(GRID,),
        in_specs=[pl.BlockSpec((BN, C_IN), lambda i: (i, 0))],
        out_specs=[pl.BlockSpec((2 * BN, 2 * C_IN), lambda i: (i, 0))],
        out_shape=[jax.ShapeDtypeStruct((2 * N, 2 * C_IN), jnp.float32)],
    )(x)[0]


def _tc1(acc1, wconv, wa1, wg1):
    return pl.pallas_call(
        _tc1_body,
        grid=(GRID,),
        in_specs=[
            pl.BlockSpec((4, BN, 2 * C_IN), lambda i: (0, i, 0)),
            pl.BlockSpec((T, C_IN, C), lambda i: (0, 0, 0)),
            pl.BlockSpec((C, CB), lambda i: (0, 0)),
            pl.BlockSpec((T, CB, CB), lambda i: (0, 0, 0)),
        ],
        out_specs=[
            pl.BlockSpec((BN, C), lambda i: (i, 0)),
            pl.BlockSpec((T, BN, CB), lambda i: (0, i, 0)),
        ],
        out_shape=[
            jax.ShapeDtypeStruct((N, C), jnp.float32),
            jax.ShapeDtypeStruct((T, N, CB), jnp.float32),
        ],
    )(acc1, wconv, wa1, wg1)


def _tc2_body(acc_ref, h1_ref, wb_ref, wa_ref, wg_ref, h2_ref, ys3_ref):
    a = (acc_ref[0] + acc_ref[1]) * (1.0 / 7.0)
    z = jax.nn.relu(a)
    z = jnp.dot(z, wb_ref[...], preferred_element_type=jnp.float32)
    h2 = jax.nn.relu(h1_ref[...] + z)
    h2_ref[...] = h2
    z3 = jax.nn.relu(jnp.dot(h2, wa_ref[...], preferred_element_type=jnp.float32))
    wg = wg_ref[...]
    for t in range(T):
        ys3_ref[t] = jnp.dot(z3, wg[t], preferred_element_type=jnp.float32)


def _tc2(acc2, h1, wb1, wa2, wg2):
    return pl.pallas_call(
        _tc2_body,
        grid=(GRID,),
        in_specs=[
            pl.BlockSpec((2, BN, CB), lambda i: (0, i, 0)),
            pl.BlockSpec((BN, C), lambda i: (i, 0)),
            pl.BlockSpec((CB, C), lambda i: (0, 0)),
            pl.BlockSpec((C, CB), lambda i: (0, 0)),
            pl.BlockSpec((T, CB, CB), lambda i: (0, 0, 0)),
        ],
        out_specs=[
            pl.BlockSpec((BN, C), lambda i: (i, 0)),
            pl.BlockSpec((T, BN, CB), lambda i: (0, i, 0)),
        ],
        out_shape=[
            jax.ShapeDtypeStruct((N, C), jnp.float32),
            jax.ShapeDtypeStruct((T, N, CB), jnp.float32),
        ],
    )(acc2, h1, wb1, wa2, wg2)


def _tc3_body(acc_ref, h2_ref, wb_ref, wp1_ref, bp1_ref, wp2_ref, bp2_ref,
              wr1_ref, br1_ref, wr2_ref, br2_ref, logits_ref, reg_ref):
    a = (acc_ref[0] + acc_ref[1]) * (1.0 / 7.0)
    z = jax.nn.relu(a)
    z = jnp.dot(z, wb_ref[...], preferred_element_type=jnp.float32)
    h3 = jax.nn.relu(h2_ref[...] + z)
    p = jax.nn.relu(jnp.dot(h3, wp1_ref[...], preferred_element_type=jnp.float32)
                    + bp1_ref[...])
    logits_ref[...] = (jnp.dot(p, wp2_ref[...], preferred_element_type=jnp.float32)
                       + bp2_ref[...])
    r = jax.nn.relu(jnp.dot(h3, wr1_ref[...], preferred_element_type=jnp.float32)
                    + br1_ref[...])
    reg_ref[...] = (jnp.dot(r, wr2_ref[...], preferred_element_type=jnp.float32)
                    + br2_ref[...])


def _tc3(acc3, h2, wb2, wp1, bp1, wp2, bp2, wr1, br1, wr2, br2):
    full = lambda *s: pl.BlockSpec(s, lambda i: tuple(0 for _ in s))
    return pl.pallas_call(
        _tc3_body,
        grid=(GRID,),
        in_specs=[
            pl.BlockSpec((2, BN, CB), lambda i: (0, i, 0)),
            pl.BlockSpec((BN, C), lambda i: (i, 0)),
            full(CB, C), full(C, H), full(1, H), full(H, 2), full(1, 2),
            full(C, H), full(1, H), full(H, 4), full(1, 4),
        ],
        out_specs=[
            pl.BlockSpec((BN, 2), lambda i: (i, 0)),
            pl.BlockSpec((BN, 4), lambda i: (i, 0)),
        ],
        out_shape=[
            jax.ShapeDtypeStruct((N, 2), jnp.float32),
            jax.ShapeDtypeStruct((N, 4), jnp.float32),
        ],
    )(acc3, h2, wb2, wp1, bp1.reshape(1, H), wp2, bp2.reshape(1, 2),
      wr1, br1.reshape(1, H), wr2, br2.reshape(1, 4))


def kernel(x, edge_index, edge_type, W_conv1, Wa1, Wg1, Wb1, Wa2, Wg2, Wb2,
           Wp1, bp1, Wp2, bp2, Wr1, br1, Wr2, br2):
    src = edge_index[0]
    dst = edge_index[1]

    pad = EP - E
    srcp = jnp.pad(src, (0, pad)).reshape(EROWS, 128)
    dstp = jnp.pad(dst, (0, pad), constant_values=N).reshape(EROWS, 128)
    typep = jnp.pad(edge_type, (0, pad)).reshape(EROWS, 128)
    g2, gx, s1a, s1b = _prep(srcp, dstp, typep)
    s1ab = jnp.concatenate([s1a, s1b], axis=0)

    zeros1 = jnp.zeros((ACC1_ROWS // 16, 2 * C_IN), jnp.float32)
    zeros23 = jnp.zeros((ACC23_ROWS // 16, CB), jnp.float32)

    xd = _mkxd(x)
    acc1 = _sc_conv1(xd, gx, s1ab, zeros1).reshape(4, N, 2 * C_IN)
    h1, ys2 = _tc1(acc1, W_conv1, Wa1, Wg1)

    acc2 = _sc_gconv(ys2.reshape(T * N, CB), g2, dstp, zeros23).reshape(2, N, CB)
    h2, ys3 = _tc2(acc2, h1, Wb1, Wa2, Wg2)

    acc3 = _sc_gconv(ys3.reshape(T * N, CB), g2, dstp, zeros23).reshape(2, N, CB)
    logits, reg = _tc3(acc3, h2, Wb2, Wp1, bp1, Wp2, bp2, Wr1, br1, Wr2, br2)
    return (logits, reg)


# speedup vs baseline: 28.4047x; 1.2940x over previous
import functools

import jax
import jax.numpy as jnp
from jax import lax
from jax.experimental import pallas as pl
from jax.experimental.pallas import tpu as pltpu
from jax.experimental.pallas import tpu_sc as plsc

N = 100000
E = 1600000
T = 7
C_IN = 4
C = 32
CB = 8
H = 32

BN = 2000
GRID = N // BN

EP = 1605632
EROWS = EP // 128
B_E = 1024
NB23 = 49
NB1 = 98
TRASH1 = 2 * N
ACC1_ROWS = 2 * N + 1088
ACC23_ROWS = N + 96
_SC_MESH = plsc.VectorSubcoreMesh(core_axis_name="c", subcore_axis_name="s")


def _prep_body(s_ref, d_ref, t_ref, g_ref, gx_ref, s1a_ref, s1b_ref):
    s = s_ref[...]
    d = d_ref[...]
    t = t_ref[...]
    g_ref[...] = t * N + s
    gx_ref[...] = 2 * s + (t & 1)
    valid = d < N
    tp = t >> 1
    trash = TRASH1 + (d & 1023)
    s1a_ref[...] = jnp.where((t < 4) & valid, tp * N + d, trash)
    s1b_ref[...] = jnp.where((t >= 4) & valid, (tp - 2) * N + d, trash)


def _prep(srcp, dstp, typep):
    spec = pl.BlockSpec((64, 128), lambda i: (i, 0))
    return pl.pallas_call(
        _prep_body,
        grid=(EROWS // 64,),
        in_specs=[spec, spec, spec],
        out_specs=[spec, spec, spec, spec],
        out_shape=[jax.ShapeDtypeStruct((EROWS, 128), jnp.int32)] * 4,
    )(srcp, dstp, typep)


def _sc_conv1_body(xd_hbm, gx2_hbm, s1a_hbm, s1b_hbm, zeros_hbm, out_hbm,
                   acc, gbuf, sbuf, rows, gsem, ssem):
    c = lax.axis_index("c")
    s = lax.axis_index("s")
    pltpu.sync_copy(zeros_hbm, acc.at[pl.ds(s * (ACC1_ROWS // 16), ACC1_ROWS // 16)])
    plsc.subcore_barrier()
    base128 = s * (NB1 * 8)

    def blk(b):
        ro = base128 + b * 8
        pltpu.sync_copy(gx2_hbm.at[pl.ds(ro, 8)], gbuf)

        @pl.when(c == 0)
        def _():
            pltpu.sync_copy(s1a_hbm.at[pl.ds(ro, 8)], sbuf)

        @pl.when(c == 1)
        def _():
            pltpu.sync_copy(s1b_hbm.at[pl.ds(ro, 8)], sbuf)
        descs = [pltpu.async_copy(xd_hbm.at[gbuf.at[j]],
                                  rows.at[pl.ds(j * 128, 128)], gsem)
                 for j in range(8)]
        for dsc in descs:
            dsc.wait()
        descs = [pltpu.async_copy(rows.at[pl.ds(j * 128, 128)],
                                  acc.at[sbuf.at[j]], ssem, add=True)
                 for j in range(8)]
        for dsc in descs:
            dsc.wait()

    pl.loop(0, NB1)(blk)
    plsc.subcore_barrier()

    @pl.when(s < 15)
    def _():
        pltpu.sync_copy(acc.at[pl.ds(s * 12504, 12504)],
                        out_hbm.at[pl.ds(c * 2 * N + s * 12504, 12504)])

    @pl.when(s == 15)
    def _():
        pltpu.sync_copy(acc.at[pl.ds(15 * 12504, 12440)],
                        out_hbm.at[pl.ds(c * 2 * N + 15 * 12504, 12440)])


_sc_conv1 = functools.partial(
    pl.kernel,
    out_type=jax.ShapeDtypeStruct((4 * N, 2 * C_IN), jnp.float32),
    mesh=_SC_MESH,
    compiler_params=pltpu.CompilerParams(use_tc_tiling_on_sc=False),
    scratch_types=[
        pltpu.VMEM_SHARED((ACC1_ROWS, 2 * C_IN), jnp.float32),
        pltpu.VMEM((8, 128), jnp.int32),
        pltpu.VMEM((8, 128), jnp.int32),
        pltpu.VMEM((B_E, 2 * C_IN), jnp.float32),
        pltpu.SemaphoreType.DMA,
        pltpu.SemaphoreType.DMA,
    ],
)(_sc_conv1_body)


def _sc_gconv_body(tab_hbm, g2_hbm, d2_hbm, zeros_hbm, out_hbm,
                   acc, gbuf, sbuf, rows, gsem, ssem):
    c = lax.axis_index("c")
    s = lax.axis_index("s")
    pltpu.sync_copy(zeros_hbm, acc.at[pl.ds(s * (ACC23_ROWS // 16), ACC23_ROWS // 16)])
    plsc.subcore_barrier()
    base128 = (c * 16 + s) * (NB23 * 8)

    def blk(b):
        ro = base128 + b * 8
        pltpu.sync_copy(g2_hbm.at[pl.ds(ro, 8)], gbuf)
        pltpu.sync_copy(d2_hbm.at[pl.ds(ro, 8)], sbuf)
        descs = [pltpu.async_copy(tab_hbm.at[gbuf.at[j]],
                                  rows.at[pl.ds(j * 128, 128)], gsem)
                 for j in range(8)]
        for dsc in descs:
            dsc.wait()
        descs = [pltpu.async_copy(rows.at[pl.ds(j * 128, 128)],
                                  acc.at[sbuf.at[j]], ssem, add=True)
                 for j in range(8)]
        for dsc in descs:
            dsc.wait()

    pl.loop(0, NB23)(blk)
    plsc.subcore_barrier()
    @pl.when(s < 15)
    def _():
        pltpu.sync_copy(acc.at[pl.ds(s * 6256, 6256)],
                        out_hbm.at[pl.ds(c * N + s * 6256, 6256)])

    @pl.when(s == 15)
    def _():
        pltpu.sync_copy(acc.at[pl.ds(15 * 6256, 6160)],
                        out_hbm.at[pl.ds(c * N + 15 * 6256, 6160)])


_sc_gconv = functools.partial(
    pl.kernel,
    out_type=jax.ShapeDtypeStruct((2 * N, CB), jnp.float32),
    mesh=_SC_MESH,
    compiler_params=pltpu.CompilerParams(use_tc_tiling_on_sc=False),
    scratch_types=[
        pltpu.VMEM_SHARED((ACC23_ROWS, CB), jnp.float32),
        pltpu.VMEM((8, 128), jnp.int32),
        pltpu.VMEM((8, 128), jnp.int32),
        pltpu.VMEM((B_E, CB), jnp.float32),
        pltpu.SemaphoreType.DMA,
        pltpu.SemaphoreType.DMA,
    ],
)(_sc_gconv_body)


def _tc1_body(acc1_ref, wconv_ref, wa1_ref, wg1_ref, h1_ref, ys2_ref):
    a = acc1_ref[...]
    w = wconv_ref[...]
    s = jnp.zeros((BN, C), dtype=jnp.float32)
    for t in range(T):
        a_t = a[t >> 1][:, (t & 1) * 4:(t & 1) * 4 + 4]
        s = s + jnp.dot(a_t, w[t], preferred_element_type=jnp.float32)
    h1 = jax.nn.relu(s * (1.0 / 7.0))
    h1_ref[...] = h1
    z2 = jax.nn.relu(jnp.dot(h1, wa1_ref[...], preferred_element_type=jnp.float32))
    wg = wg1_ref[...]
    for t in range(T):
        ys2_ref[t] = jnp.dot(z2, wg[t], preferred_element_type=jnp.float32)


def _mkxd_body(x_ref, xd_ref):
    xr = x_ref[...]
    z = jnp.zeros((BN, C_IN), jnp.float32)
    a = jnp.concatenate([xr, z], axis=1)
    b = jnp.concatenate([z, xr], axis=1)
    xd_ref[...] = jnp.stack([a, b], axis=1).reshape(2 * BN, 2 * C_IN)


def _mkxd(x):
    return pl.pallas_call(
        _mkxd_body,
        grid=(GRID,),
        in_specs=[pl.BlockSpec((BN, C_IN), lambda i: (i, 0))],
        out_specs=[pl.BlockSpec((2 * BN, 2 * C_IN), lambda i: (i, 0))],
        out_shape=[jax.ShapeDtypeStruct((2 * N, 2 * C_IN), jnp.float32)],
    )(x)[0]


def _tc1(acc1, wconv, wa1, wg1):
    return pl.pallas_call(
        _tc1_body,
        grid=(GRID,),
        in_specs=[
            pl.BlockSpec((4, BN, 2 * C_IN), lambda i: (0, i, 0)),
            pl.BlockSpec((T, C_IN, C), lambda i: (0, 0, 0)),
            pl.BlockSpec((C, CB), lambda i: (0, 0)),
            pl.BlockSpec((T, CB, CB), lambda i: (0, 0, 0)),
        ],
        out_specs=[
            pl.BlockSpec((BN, C), lambda i: (i, 0)),
            pl.BlockSpec((T, BN, CB), lambda i: (0, i, 0)),
        ],
        out_shape=[
            jax.ShapeDtypeStruct((N, C), jnp.float32),
            jax.ShapeDtypeStruct((T, N, CB), jnp.float32),
        ],
    )(acc1, wconv, wa1, wg1)


def _tc2_body(acc_ref, h1_ref, wb_ref, wa_ref, wg_ref, h2_ref, ys3_ref):
    a = (acc_ref[0] + acc_ref[1]) * (1.0 / 7.0)
    z = jax.nn.relu(a)
    z = jnp.dot(z, wb_ref[...], preferred_element_type=jnp.float32)
    h2 = jax.nn.relu(h1_ref[...] + z)
    h2_ref[...] = h2
    z3 = jax.nn.relu(jnp.dot(h2, wa_ref[...], preferred_element_type=jnp.float32))
    wg = wg_ref[...]
    for t in range(T):
        ys3_ref[t] = jnp.dot(z3, wg[t], preferred_element_type=jnp.float32)


def _tc2(acc2, h1, wb1, wa2, wg2):
    return pl.pallas_call(
        _tc2_body,
        grid=(GRID,),
        in_specs=[
            pl.BlockSpec((2, BN, CB), lambda i: (0, i, 0)),
            pl.BlockSpec((BN, C), lambda i: (i, 0)),
            pl.BlockSpec((CB, C), lambda i: (0, 0)),
            pl.BlockSpec((C, CB), lambda i: (0, 0)),
            pl.BlockSpec((T, CB, CB), lambda i: (0, 0, 0)),
        ],
        out_specs=[
            pl.BlockSpec((BN, C), lambda i: (i, 0)),
            pl.BlockSpec((T, BN, CB), lambda i: (0, i, 0)),
        ],
        out_shape=[
            jax.ShapeDtypeStruct((N, C), jnp.float32),
            jax.ShapeDtypeStruct((T, N, CB), jnp.float32),
        ],
    )(acc2, h1, wb1, wa2, wg2)


def _tc3_body(acc_ref, h2_ref, wb_ref, wp1_ref, bp1_ref, wp2_ref, bp2_ref,
              wr1_ref, br1_ref, wr2_ref, br2_ref, logits_ref, reg_ref):
    a = (acc_ref[0] + acc_ref[1]) * (1.0 / 7.0)
    z = jax.nn.relu(a)
    z = jnp.dot(z, wb_ref[...], preferred_element_type=jnp.float32)
    h3 = jax.nn.relu(h2_ref[...] + z)
    p = jax.nn.relu(jnp.dot(h3, wp1_ref[...], preferred_element_type=jnp.float32)
                    + bp1_ref[...])
    logits_ref[...] = (jnp.dot(p, wp2_ref[...], preferred_element_type=jnp.float32)
                       + bp2_ref[...])
    r = jax.nn.relu(jnp.dot(h3, wr1_ref[...], preferred_element_type=jnp.float32)
                    + br1_ref[...])
    reg_ref[...] = (jnp.dot(r, wr2_ref[...], preferred_element_type=jnp.float32)
                    + br2_ref[...])


def _tc3(acc3, h2, wb2, wp1, bp1, wp2, bp2, wr1, br1, wr2, br2):
    full = lambda *s: pl.BlockSpec(s, lambda i: tuple(0 for _ in s))
    return pl.pallas_call(
        _tc3_body,
        grid=(GRID,),
        in_specs=[
            pl.BlockSpec((2, BN, CB), lambda i: (0, i, 0)),
            pl.BlockSpec((BN, C), lambda i: (i, 0)),
            full(CB, C), full(C, H), full(1, H), full(H, 2), full(1, 2),
            full(C, H), full(1, H), full(H, 4), full(1, 4),
        ],
        out_specs=[
            pl.BlockSpec((BN, 2), lambda i: (i, 0)),
            pl.BlockSpec((BN, 4), lambda i: (i, 0)),
        ],
        out_shape=[
            jax.ShapeDtypeStruct((N, 2), jnp.float32),
            jax.ShapeDtypeStruct((N, 4), jnp.float32),
        ],
    )(acc3, h2, wb2, wp1, bp1.reshape(1, H), wp2, bp2.reshape(1, 2),
      wr1, br1.reshape(1, H), wr2, br2.reshape(1, 4))


def kernel(x, edge_index, edge_type, W_conv1, Wa1, Wg1, Wb1, Wa2, Wg2, Wb2,
           Wp1, bp1, Wp2, bp2, Wr1, br1, Wr2, br2):
    src = edge_index[0]
    dst = edge_index[1]

    pad = EP - E
    srcp = jnp.pad(src, (0, pad)).reshape(EROWS, 128)
    dstp = jnp.pad(dst, (0, pad), constant_values=N).reshape(EROWS, 128)
    typep = jnp.pad(edge_type, (0, pad)).reshape(EROWS, 128)
    g2, gx, s1a, s1b = _prep(srcp, dstp, typep)

    zeros1 = jnp.zeros((ACC1_ROWS // 16, 2 * C_IN), jnp.float32)
    zeros23 = jnp.zeros((ACC23_ROWS // 16, CB), jnp.float32)

    xd = _mkxd(x)
    acc1 = _sc_conv1(xd, gx, s1a, s1b, zeros1).reshape(4, N, 2 * C_IN)
    h1, ys2 = _tc1(acc1, W_conv1, Wa1, Wg1)

    acc2 = _sc_gconv(ys2.reshape(T * N, CB), g2, dstp, zeros23).reshape(2, N, CB)
    h2, ys3 = _tc2(acc2, h1, Wb1, Wa2, Wg2)

    acc3 = _sc_gconv(ys3.reshape(T * N, CB), g2, dstp, zeros23).reshape(2, N, CB)
    logits, reg = _tc3(acc3, h2, Wb2, Wp1, bp1, Wp2, bp2, Wr1, br1, Wr2, br2)
    return (logits, reg)


# speedup vs baseline: 29.1736x; 1.0271x over previous
import functools

import jax
import jax.numpy as jnp
from jax import lax
from jax.experimental import pallas as pl
from jax.experimental.pallas import tpu as pltpu
from jax.experimental.pallas import tpu_sc as plsc

N = 100000
E = 1600000
T = 7
C_IN = 4
C = 32
CB = 8
H = 32

BN = 2000
GRID = N // BN

EP = 1605632
EROWS = EP // 128
B_E = 1024
NB23 = 49
NB1 = 98
TRASH1 = 2 * N
ACC1_ROWS = 2 * N + 1088
ACC23_ROWS = N + 96
_SC_MESH = plsc.VectorSubcoreMesh(core_axis_name="c", subcore_axis_name="s")


def _prep_body(s_ref, d_ref, t_ref, g_ref, gx_ref, s1a_ref, s1b_ref):
    s = s_ref[...]
    d = d_ref[...]
    t = t_ref[...]
    g_ref[...] = t * N + s
    gx_ref[...] = 2 * s + (t & 1)
    valid = d < N
    tp = t >> 1
    trash = TRASH1 + (d & 1023)
    s1a_ref[...] = jnp.where((t < 4) & valid, tp * N + d, trash)
    s1b_ref[...] = jnp.where((t >= 4) & valid, (tp - 2) * N + d, trash)


def _prep(srcp, dstp, typep):
    spec = pl.BlockSpec((64, 128), lambda i: (i, 0))
    return pl.pallas_call(
        _prep_body,
        grid=(EROWS // 64,),
        in_specs=[spec, spec, spec],
        out_specs=[spec, spec, spec, spec],
        out_shape=[jax.ShapeDtypeStruct((EROWS, 128), jnp.int32)] * 4,
    )(srcp, dstp, typep)


def _gs_pipeline(nb, loadidx, table, acc, sets):
    def drain_scatters(sb, rw, ss):
        for j in range(8):
            pltpu.make_async_copy(rw.at[pl.ds(j * 128, 128)],
                                  acc.at[sb.at[j]], ss).wait()

    def do_block(b_idx, gb, sb, rw, gs, ss):
        loadidx(b_idx, gb, sb)
        descs = [pltpu.async_copy(table.at[gb.at[j]],
                                  rw.at[pl.ds(j * 128, 128)], gs)
                 for j in range(8)]
        for dsc in descs:
            dsc.wait()
        for j in range(8):
            pltpu.async_copy(rw.at[pl.ds(j * 128, 128)],
                             acc.at[sb.at[j]], ss, add=True)

    def pair(p):
        for half in (0, 1):
            gb, sb, rw, gs, ss = sets[half]
            b = 2 * p + half

            def step():
                @pl.when(p > 0)
                def _():
                    drain_scatters(sb, rw, ss)
                do_block(b, gb, sb, rw, gs, ss)

            if nb % 2 == 1 and half == 1:
                pl.when(b < nb)(step)
            else:
                step()

    pl.loop(0, (nb + 1) // 2)(pair)
    for half in (0, 1):
        gb, sb, rw, gs, ss = sets[half]
        drain_scatters(sb, rw, ss)

def _sc_conv1_body(xd_hbm, gx2_hbm, s1a_hbm, s1b_hbm, zeros_hbm, out_hbm,
                   acc, gbuf, sbuf, rows, gsem, ssem,
                   gbuf2, sbuf2, rows2, gsem2, ssem2):
    c = lax.axis_index("c")
    s = lax.axis_index("s")
    pltpu.sync_copy(zeros_hbm, acc.at[pl.ds(s * (ACC1_ROWS // 16), ACC1_ROWS // 16)])
    plsc.subcore_barrier()
    base128 = s * (NB1 * 8)

    def loadidx(b, gb, sb):
        ro = base128 + b * 8
        pltpu.sync_copy(gx2_hbm.at[pl.ds(ro, 8)], gb)

        @pl.when(c == 0)
        def _():
            pltpu.sync_copy(s1a_hbm.at[pl.ds(ro, 8)], sb)

        @pl.when(c == 1)
        def _():
            pltpu.sync_copy(s1b_hbm.at[pl.ds(ro, 8)], sb)

    _gs_pipeline(NB1, loadidx, xd_hbm, acc,
                 [(gbuf, sbuf, rows, gsem, ssem),
                  (gbuf2, sbuf2, rows2, gsem2, ssem2)])
    plsc.subcore_barrier()

    @pl.when(s < 15)
    def _():
        pltpu.sync_copy(acc.at[pl.ds(s * 12504, 12504)],
                        out_hbm.at[pl.ds(c * 2 * N + s * 12504, 12504)])

    @pl.when(s == 15)
    def _():
        pltpu.sync_copy(acc.at[pl.ds(15 * 12504, 12440)],
                        out_hbm.at[pl.ds(c * 2 * N + 15 * 12504, 12440)])


_sc_conv1 = functools.partial(
    pl.kernel,
    out_type=jax.ShapeDtypeStruct((4 * N, 2 * C_IN), jnp.float32),
    mesh=_SC_MESH,
    compiler_params=pltpu.CompilerParams(use_tc_tiling_on_sc=False),
    scratch_types=[
        pltpu.VMEM_SHARED((ACC1_ROWS, 2 * C_IN), jnp.float32),
        pltpu.VMEM((8, 128), jnp.int32),
        pltpu.VMEM((8, 128), jnp.int32),
        pltpu.VMEM((B_E, 2 * C_IN), jnp.float32),
        pltpu.SemaphoreType.DMA,
        pltpu.SemaphoreType.DMA,
        pltpu.VMEM((8, 128), jnp.int32),
        pltpu.VMEM((8, 128), jnp.int32),
        pltpu.VMEM((B_E, 2 * C_IN), jnp.float32),
        pltpu.SemaphoreType.DMA,
        pltpu.SemaphoreType.DMA,
    ],
)(_sc_conv1_body)


def _sc_gconv_body(tab_hbm, g2_hbm, d2_hbm, zeros_hbm, out_hbm,
                   acc, gbuf, sbuf, rows, gsem, ssem,
                   gbuf2, sbuf2, rows2, gsem2, ssem2):
    c = lax.axis_index("c")
    s = lax.axis_index("s")
    pltpu.sync_copy(zeros_hbm, acc.at[pl.ds(s * (ACC23_ROWS // 16), ACC23_ROWS // 16)])
    plsc.subcore_barrier()
    base128 = (c * 16 + s) * (NB23 * 8)

    def loadidx(b, gb, sb):
        ro = base128 + b * 8
        pltpu.sync_copy(g2_hbm.at[pl.ds(ro, 8)], gb)
        pltpu.sync_copy(d2_hbm.at[pl.ds(ro, 8)], sb)

    _gs_pipeline(NB23, loadidx, tab_hbm, acc,
                 [(gbuf, sbuf, rows, gsem, ssem),
                  (gbuf2, sbuf2, rows2, gsem2, ssem2)])
    plsc.subcore_barrier()
    @pl.when(s < 15)
    def _():
        pltpu.sync_copy(acc.at[pl.ds(s * 6256, 6256)],
                        out_hbm.at[pl.ds(c * N + s * 6256, 6256)])

    @pl.when(s == 15)
    def _():
        pltpu.sync_copy(acc.at[pl.ds(15 * 6256, 6160)],
                        out_hbm.at[pl.ds(c * N + 15 * 6256, 6160)])


_sc_gconv = functools.partial(
    pl.kernel,
    out_type=jax.ShapeDtypeStruct((2 * N, CB), jnp.float32),
    mesh=_SC_MESH,
    compiler_params=pltpu.CompilerParams(use_tc_tiling_on_sc=False),
    scratch_types=[
        pltpu.VMEM_SHARED((ACC23_ROWS, CB), jnp.float32),
        pltpu.VMEM((8, 128), jnp.int32),
        pltpu.VMEM((8, 128), jnp.int32),
        pltpu.VMEM((B_E, CB), jnp.float32),
        pltpu.SemaphoreType.DMA,
        pltpu.SemaphoreType.DMA,
        pltpu.VMEM((8, 128), jnp.int32),
        pltpu.VMEM((8, 128), jnp.int32),
        pltpu.VMEM((B_E, CB), jnp.float32),
        pltpu.SemaphoreType.DMA,
        pltpu.SemaphoreType.DMA,
    ],
)(_sc_gconv_body)


def _tc1_body(acc1_ref, wconv_ref, wa1_ref, wg1_ref, h1_ref, ys2_ref):
    a = acc1_ref[...]
    w = wconv_ref[...]
    s = jnp.zeros((BN, C), dtype=jnp.float32)
    for t in range(T):
        a_t = a[t >> 1][:, (t & 1) * 4:(t & 1) * 4 + 4]
        s = s + jnp.dot(a_t, w[t], preferred_element_type=jnp.float32)
    h1 = jax.nn.relu(s * (1.0 / 7.0))
    h1_ref[...] = h1
    z2 = jax.nn.relu(jnp.dot(h1, wa1_ref[...], preferred_element_type=jnp.float32))
    wg = wg1_ref[...]
    for t in range(T):
        ys2_ref[t] = jnp.dot(z2, wg[t], preferred_element_type=jnp.float32)


def _mkxd_body(x_ref, xd_ref):
    xr = x_ref[...]
    z = jnp.zeros((BN, C_IN), jnp.float32)
    a = jnp.concatenate([xr, z], axis=1)
    b = jnp.concatenate([z, xr], axis=1)
    xd_ref[...] = jnp.stack([a, b], axis=1).reshape(2 * BN, 2 * C_IN)


def _mkxd(x):
    return pl.pallas_call(
        _mkxd_body,
        grid=(GRID,),
        in_specs=[pl.BlockSpec((BN, C_IN), lambda i: (i, 0))],
        out_specs=[pl.BlockSpec((2 * BN, 2 * C_IN), lambda i: (i, 0))],
        out_shape=[jax.ShapeDtypeStruct((2 * N, 2 * C_IN), jnp.float32)],
    )(x)[0]


def _tc1(acc1, wconv, wa1, wg1):
    return pl.pallas_call(
        _tc1_body,
        grid=(GRID,),
        in_specs=[
            pl.BlockSpec((4, BN, 2 * C_IN), lambda i: (0, i, 0)),
            pl.BlockSpec((T, C_IN, C), lambda i: (0, 0, 0)),
            pl.BlockSpec((C, CB), lambda i: (0, 0)),
            pl.BlockSpec((T, CB, CB), lambda i: (0, 0, 0)),
        ],
        out_specs=[
            pl.BlockSpec((BN, C), lambda i: (i, 0)),
            pl.BlockSpec((T, BN, CB), lambda i: (0, i, 0)),
        ],
        out_shape=[
            jax.ShapeDtypeStruct((N, C), jnp.float32),
            jax.ShapeDtypeStruct((T, N, CB), jnp.float32),
        ],
    )(acc1, wconv, wa1, wg1)


def _tc2_body(acc_ref, h1_ref, wb_ref, wa_ref, wg_ref, h2_ref, ys3_ref):
    a = (acc_ref[0] + acc_ref[1]) * (1.0 / 7.0)
    z = jax.nn.relu(a)
    z = jnp.dot(z, wb_ref[...], preferred_element_type=jnp.float32)
    h2 = jax.nn.relu(h1_ref[...] + z)
    h2_ref[...] = h2
    z3 = jax.nn.relu(jnp.dot(h2, wa_ref[...], preferred_element_type=jnp.float32))
    wg = wg_ref[...]
    for t in range(T):
        ys3_ref[t] = jnp.dot(z3, wg[t], preferred_element_type=jnp.float32)


def _tc2(acc2, h1, wb1, wa2, wg2):
    return pl.pallas_call(
        _tc2_body,
        grid=(GRID,),
        in_specs=[
            pl.BlockSpec((2, BN, CB), lambda i: (0, i, 0)),
            pl.BlockSpec((BN, C), lambda i: (i, 0)),
            pl.BlockSpec((CB, C), lambda i: (0, 0)),
            pl.BlockSpec((C, CB), lambda i: (0, 0)),
            pl.BlockSpec((T, CB, CB), lambda i: (0, 0, 0)),
        ],
        out_specs=[
            pl.BlockSpec((BN, C), lambda i: (i, 0)),
            pl.BlockSpec((T, BN, CB), lambda i: (0, i, 0)),
        ],
        out_shape=[
            jax.ShapeDtypeStruct((N, C), jnp.float32),
            jax.ShapeDtypeStruct((T, N, CB), jnp.float32),
        ],
    )(acc2, h1, wb1, wa2, wg2)


def _tc3_body(acc_ref, h2_ref, wb_ref, wp1_ref, bp1_ref, wp2_ref, bp2_ref,
              wr1_ref, br1_ref, wr2_ref, br2_ref, logits_ref, reg_ref):
    a = (acc_ref[0] + acc_ref[1]) * (1.0 / 7.0)
    z = jax.nn.relu(a)
    z = jnp.dot(z, wb_ref[...], preferred_element_type=jnp.float32)
    h3 = jax.nn.relu(h2_ref[...] + z)
    p = jax.nn.relu(jnp.dot(h3, wp1_ref[...], preferred_element_type=jnp.float32)
                    + bp1_ref[...])
    logits_ref[...] = (jnp.dot(p, wp2_ref[...], preferred_element_type=jnp.float32)
                       + bp2_ref[...])
    r = jax.nn.relu(jnp.dot(h3, wr1_ref[...], preferred_element_type=jnp.float32)
                    + br1_ref[...])
    reg_ref[...] = (jnp.dot(r, wr2_ref[...], preferred_element_type=jnp.float32)
                    + br2_ref[...])


def _tc3(acc3, h2, wb2, wp1, bp1, wp2, bp2, wr1, br1, wr2, br2):
    full = lambda *s: pl.BlockSpec(s, lambda i: tuple(0 for _ in s))
    return pl.pallas_call(
        _tc3_body,
        grid=(GRID,),
        in_specs=[
            pl.BlockSpec((2, BN, CB), lambda i: (0, i, 0)),
            pl.BlockSpec((BN, C), lambda i: (i, 0)),
            full(CB, C), full(C, H), full(1, H), full(H, 2), full(1, 2),
            full(C, H), full(1, H), full(H, 4), full(1, 4),
        ],
        out_specs=[
            pl.BlockSpec((BN, 2), lambda i: (i, 0)),
            pl.BlockSpec((BN, 4), lambda i: (i, 0)),
        ],
        out_shape=[
            jax.ShapeDtypeStruct((N, 2), jnp.float32),
            jax.ShapeDtypeStruct((N, 4), jnp.float32),
        ],
    )(acc3, h2, wb2, wp1, bp1.reshape(1, H), wp2, bp2.reshape(1, 2),
      wr1, br1.reshape(1, H), wr2, br2.reshape(1, 4))


def kernel(x, edge_index, edge_type, W_conv1, Wa1, Wg1, Wb1, Wa2, Wg2, Wb2,
           Wp1, bp1, Wp2, bp2, Wr1, br1, Wr2, br2):
    src = edge_index[0]
    dst = edge_index[1]

    pad = EP - E
    srcp = jnp.pad(src, (0, pad)).reshape(EROWS, 128)
    dstp = jnp.pad(dst, (0, pad), constant_values=N).reshape(EROWS, 128)
    typep = jnp.pad(edge_type, (0, pad)).reshape(EROWS, 128)
    g2, gx, s1a, s1b = _prep(srcp, dstp, typep)

    zeros1 = jnp.zeros((ACC1_ROWS // 16, 2 * C_IN), jnp.float32)
    zeros23 = jnp.zeros((ACC23_ROWS // 16, CB), jnp.float32)

    xd = _mkxd(x)
    acc1 = _sc_conv1(xd, gx, s1a, s1b, zeros1).reshape(4, N, 2 * C_IN)
    h1, ys2 = _tc1(acc1, W_conv1, Wa1, Wg1)

    acc2 = _sc_gconv(ys2.reshape(T * N, CB), g2, dstp, zeros23).reshape(2, N, CB)
    h2, ys3 = _tc2(acc2, h1, Wb1, Wa2, Wg2)

    acc3 = _sc_gconv(ys3.reshape(T * N, CB), g2, dstp, zeros23).reshape(2, N, CB)
    logits, reg = _tc3(acc3, h2, Wb2, Wp1, bp1, Wp2, bp2, Wr1, br1, Wr2, br2)
    return (logits, reg)


# speedup vs baseline: 40.4315x; 1.3859x over previous
import functools

import jax
import jax.numpy as jnp
from jax import lax
from jax.experimental import pallas as pl
from jax.experimental.pallas import tpu as pltpu
from jax.experimental.pallas import tpu_sc as plsc

N = 100000
E = 1600000
T = 7
C_IN = 4
C = 32
CB = 8
H = 32

BN = 2000
GRID = N // BN

EP = 1605632
EROWS = EP // 128
B_E = 1024
NB23 = 49
NB1 = 98
TRASH1 = 2 * N
ACC1_ROWS = 2 * N + 1088
ACC23_ROWS = N + 96
_SC_MESH = plsc.VectorSubcoreMesh(core_axis_name="c", subcore_axis_name="s")


def _prep_body(s_ref, d_ref, t_ref, g_ref, gx_ref, s1a_ref, s1b_ref):
    s = s_ref[...]
    d = d_ref[...]
    t = t_ref[...]
    g_ref[...] = 16 * s + t
    gx_ref[...] = 2 * s + (t & 1)
    valid = d < N
    tp = t >> 1
    trash = TRASH1 + (d & 1023)
    s1a_ref[...] = jnp.where((t < 4) & valid, tp * N + d, trash)
    s1b_ref[...] = jnp.where((t >= 4) & valid, (tp - 2) * N + d, trash)


def _prep(srcp, dstp, typep):
    spec = pl.BlockSpec((64, 128), lambda i: (i, 0))
    return pl.pallas_call(
        _prep_body,
        grid=(EROWS // 64,),
        in_specs=[spec, spec, spec],
        out_specs=[spec, spec, spec, spec],
        out_shape=[jax.ShapeDtypeStruct((EROWS, 128), jnp.int32)] * 4,
    )(srcp, dstp, typep)


def _gs_pipeline(nb, loadidx, table, acc, sets):
    def drain_scatters(sb, rw, ss):
        for j in range(8):
            pltpu.make_async_copy(rw.at[pl.ds(j * 128, 128)],
                                  acc.at[sb.at[j]], ss).wait()

    def do_block(b_idx, gb, sb, rw, gs, ss):
        loadidx(b_idx, gb, sb)
        descs = [pltpu.async_copy(table.at[gb.at[j]],
                                  rw.at[pl.ds(j * 128, 128)], gs)
                 for j in range(8)]
        for dsc in descs:
            dsc.wait()
        for j in range(8):
            pltpu.async_copy(rw.at[pl.ds(j * 128, 128)],
                             acc.at[sb.at[j]], ss, add=True)

    def pair(p):
        for half in (0, 1):
            gb, sb, rw, gs, ss = sets[half]
            b = 2 * p + half

            def step():
                @pl.when(p > 0)
                def _():
                    drain_scatters(sb, rw, ss)
                do_block(b, gb, sb, rw, gs, ss)

            if nb % 2 == 1 and half == 1:
                pl.when(b < nb)(step)
            else:
                step()

    pl.loop(0, (nb + 1) // 2)(pair)
    for half in (0, 1):
        gb, sb, rw, gs, ss = sets[half]
        drain_scatters(sb, rw, ss)

def _sc_conv1_body(xd_hbm, gx2_hbm, s1a_hbm, s1b_hbm, zeros_hbm, out_hbm,
                   acc, gbuf, sbuf, rows, gsem, ssem,
                   gbuf2, sbuf2, rows2, gsem2, ssem2):
    c = lax.axis_index("c")
    s = lax.axis_index("s")
    pltpu.sync_copy(zeros_hbm, acc.at[pl.ds(s * (ACC1_ROWS // 16), ACC1_ROWS // 16)])
    plsc.subcore_barrier()
    base128 = s * (NB1 * 8)

    def loadidx(b, gb, sb):
        ro = base128 + b * 8
        pltpu.sync_copy(gx2_hbm.at[pl.ds(ro, 8)], gb)

        @pl.when(c == 0)
        def _():
            pltpu.sync_copy(s1a_hbm.at[pl.ds(ro, 8)], sb)

        @pl.when(c == 1)
        def _():
            pltpu.sync_copy(s1b_hbm.at[pl.ds(ro, 8)], sb)

    _gs_pipeline(NB1, loadidx, xd_hbm, acc,
                 [(gbuf, sbuf, rows, gsem, ssem),
                  (gbuf2, sbuf2, rows2, gsem2, ssem2)])
    plsc.subcore_barrier()

    @pl.when(s < 15)
    def _():
        pltpu.sync_copy(acc.at[pl.ds(s * 12504, 12504)],
                        out_hbm.at[pl.ds(c * 2 * N + s * 12504, 12504)])

    @pl.when(s == 15)
    def _():
        pltpu.sync_copy(acc.at[pl.ds(15 * 12504, 12440)],
                        out_hbm.at[pl.ds(c * 2 * N + 15 * 12504, 12440)])


_sc_conv1 = functools.partial(
    pl.kernel,
    out_type=jax.ShapeDtypeStruct((4 * N, 2 * C_IN), jnp.float32),
    mesh=_SC_MESH,
    compiler_params=pltpu.CompilerParams(use_tc_tiling_on_sc=False),
    scratch_types=[
        pltpu.VMEM_SHARED((ACC1_ROWS, 2 * C_IN), jnp.float32),
        pltpu.VMEM((8, 128), jnp.int32),
        pltpu.VMEM((8, 128), jnp.int32),
        pltpu.VMEM((B_E, 2 * C_IN), jnp.float32),
        pltpu.SemaphoreType.DMA,
        pltpu.SemaphoreType.DMA,
        pltpu.VMEM((8, 128), jnp.int32),
        pltpu.VMEM((8, 128), jnp.int32),
        pltpu.VMEM((B_E, 2 * C_IN), jnp.float32),
        pltpu.SemaphoreType.DMA,
        pltpu.SemaphoreType.DMA,
    ],
)(_sc_conv1_body)


def _sc_gconv_body(tab_hbm, g2_hbm, d2_hbm, zeros_hbm, out_hbm,
                   acc, gbuf, sbuf, rows, gsem, ssem,
                   gbuf2, sbuf2, rows2, gsem2, ssem2):
    c = lax.axis_index("c")
    s = lax.axis_index("s")
    pltpu.sync_copy(zeros_hbm, acc.at[pl.ds(s * (ACC23_ROWS // 16), ACC23_ROWS // 16)])
    plsc.subcore_barrier()
    base128 = (c * 16 + s) * (NB23 * 8)

    def loadidx(b, gb, sb):
        ro = base128 + b * 8
        pltpu.sync_copy(g2_hbm.at[pl.ds(ro, 8)], gb)
        pltpu.sync_copy(d2_hbm.at[pl.ds(ro, 8)], sb)

    _gs_pipeline(NB23, loadidx, tab_hbm, acc,
                 [(gbuf, sbuf, rows, gsem, ssem),
                  (gbuf2, sbuf2, rows2, gsem2, ssem2)])
    plsc.subcore_barrier()
    @pl.when(s < 15)
    def _():
        pltpu.sync_copy(acc.at[pl.ds(s * 6256, 6256)],
                        out_hbm.at[pl.ds(c * N + s * 6256, 6256)])

    @pl.when(s == 15)
    def _():
        pltpu.sync_copy(acc.at[pl.ds(15 * 6256, 6160)],
                        out_hbm.at[pl.ds(c * N + 15 * 6256, 6160)])


_sc_gconv = functools.partial(
    pl.kernel,
    out_type=jax.ShapeDtypeStruct((2 * N, CB), jnp.float32),
    mesh=_SC_MESH,
    compiler_params=pltpu.CompilerParams(use_tc_tiling_on_sc=False),
    scratch_types=[
        pltpu.VMEM_SHARED((ACC23_ROWS, CB), jnp.float32),
        pltpu.VMEM((8, 128), jnp.int32),
        pltpu.VMEM((8, 128), jnp.int32),
        pltpu.VMEM((B_E, CB), jnp.float32),
        pltpu.SemaphoreType.DMA,
        pltpu.SemaphoreType.DMA,
        pltpu.VMEM((8, 128), jnp.int32),
        pltpu.VMEM((8, 128), jnp.int32),
        pltpu.VMEM((B_E, CB), jnp.float32),
        pltpu.SemaphoreType.DMA,
        pltpu.SemaphoreType.DMA,
    ],
)(_sc_gconv_body)


def _tc1_body(acc1_ref, wbig_ref, wa1_ref, wgw_ref, h1_ref, ys2_ref):
    a = acc1_ref[...]
    wbig = wbig_ref[...]
    s = jnp.zeros((BN, C), dtype=jnp.float32)
    for p in range(4):
        s = s + jnp.dot(a[p], wbig[p * 8:(p + 1) * 8],
                        preferred_element_type=jnp.float32)
    h1 = jax.nn.relu(s * (1.0 / 7.0))
    h1_ref[...] = h1
    z2 = jax.nn.relu(jnp.dot(h1, wa1_ref[...], preferred_element_type=jnp.float32))
    ys2_ref[...] = jnp.dot(z2, wgw_ref[...], preferred_element_type=jnp.float32)


def _mkxd_body(x_ref, xd_ref):
    xr = x_ref[...]
    z = jnp.zeros((BN, C_IN), jnp.float32)
    a = jnp.concatenate([xr, z], axis=1)
    b = jnp.concatenate([z, xr], axis=1)
    xd_ref[...] = jnp.stack([a, b], axis=1).reshape(2 * BN, 2 * C_IN)


def _mkxd(x):
    return pl.pallas_call(
        _mkxd_body,
        grid=(GRID,),
        in_specs=[pl.BlockSpec((BN, C_IN), lambda i: (i, 0))],
        out_specs=[pl.BlockSpec((2 * BN, 2 * C_IN), lambda i: (i, 0))],
        out_shape=[jax.ShapeDtypeStruct((2 * N, 2 * C_IN), jnp.float32)],
    )(x)[0]


def _tc1(acc1, wconv, wa1, wg1):
    return pl.pallas_call(
        _tc1_body,
        grid=(GRID,),
        in_specs=[
            pl.BlockSpec((4, BN, 2 * C_IN), lambda i: (0, i, 0)),
            pl.BlockSpec((C, C), lambda i: (0, 0)),
            pl.BlockSpec((C, CB), lambda i: (0, 0)),
            pl.BlockSpec((CB, 128), lambda i: (0, 0)),
        ],
        out_specs=[
            pl.BlockSpec((BN, C), lambda i: (i, 0)),
            pl.BlockSpec((BN, 128), lambda i: (i, 0)),
        ],
        out_shape=[
            jax.ShapeDtypeStruct((N, C), jnp.float32),
            jax.ShapeDtypeStruct((N, 128), jnp.float32),
        ],
    )(acc1, wconv, wa1, wg1)


def _tc2_body(acc_ref, h1_ref, wb_ref, wa_ref, wg_ref, h2_ref, ys3_ref):
    a = (acc_ref[0] + acc_ref[1]) * (1.0 / 7.0)
    z = jax.nn.relu(a)
    z = jnp.dot(z, wb_ref[...], preferred_element_type=jnp.float32)
    h2 = jax.nn.relu(h1_ref[...] + z)
    h2_ref[...] = h2
    z3 = jax.nn.relu(jnp.dot(h2, wa_ref[...], preferred_element_type=jnp.float32))
    ys3_ref[...] = jnp.dot(z3, wg_ref[...], preferred_element_type=jnp.float32)


def _tc2(acc2, h1, wb1, wa2, wg2):
    return pl.pallas_call(
        _tc2_body,
        grid=(GRID,),
        in_specs=[
            pl.BlockSpec((2, BN, CB), lambda i: (0, i, 0)),
            pl.BlockSpec((BN, C), lambda i: (i, 0)),
            pl.BlockSpec((CB, C), lambda i: (0, 0)),
            pl.BlockSpec((C, CB), lambda i: (0, 0)),
            pl.BlockSpec((CB, 128), lambda i: (0, 0)),
        ],
        out_specs=[
            pl.BlockSpec((BN, C), lambda i: (i, 0)),
            pl.BlockSpec((BN, 128), lambda i: (i, 0)),
        ],
        out_shape=[
            jax.ShapeDtypeStruct((N, C), jnp.float32),
            jax.ShapeDtypeStruct((N, 128), jnp.float32),
        ],
    )(acc2, h1, wb1, wa2, wg2)


def _tc3_body(acc_ref, h2_ref, wb_ref, wp1_ref, bp1_ref, wp2_ref, bp2_ref,
              wr1_ref, br1_ref, wr2_ref, br2_ref, logits_ref, reg_ref):
    a = (acc_ref[0] + acc_ref[1]) * (1.0 / 7.0)
    z = jax.nn.relu(a)
    z = jnp.dot(z, wb_ref[...], preferred_element_type=jnp.float32)
    h3 = jax.nn.relu(h2_ref[...] + z)
    p = jax.nn.relu(jnp.dot(h3, wp1_ref[...], preferred_element_type=jnp.float32)
                    + bp1_ref[...])
    logits_ref[...] = (jnp.dot(p, wp2_ref[...], preferred_element_type=jnp.float32)
                       + bp2_ref[...])
    r = jax.nn.relu(jnp.dot(h3, wr1_ref[...], preferred_element_type=jnp.float32)
                    + br1_ref[...])
    reg_ref[...] = (jnp.dot(r, wr2_ref[...], preferred_element_type=jnp.float32)
                    + br2_ref[...])


def _tc3(acc3, h2, wb2, wp1, bp1, wp2, bp2, wr1, br1, wr2, br2):
    full = lambda *s: pl.BlockSpec(s, lambda i: tuple(0 for _ in s))
    return pl.pallas_call(
        _tc3_body,
        grid=(GRID,),
        in_specs=[
            pl.BlockSpec((2, BN, CB), lambda i: (0, i, 0)),
            pl.BlockSpec((BN, C), lambda i: (i, 0)),
            full(CB, C), full(C, H), full(1, H), full(H, 2), full(1, 2),
            full(C, H), full(1, H), full(H, 4), full(1, 4),
        ],
        out_specs=[
            pl.BlockSpec((BN, 2), lambda i: (i, 0)),
            pl.BlockSpec((BN, 4), lambda i: (i, 0)),
        ],
        out_shape=[
            jax.ShapeDtypeStruct((N, 2), jnp.float32),
            jax.ShapeDtypeStruct((N, 4), jnp.float32),
        ],
    )(acc3, h2, wb2, wp1, bp1.reshape(1, H), wp2, bp2.reshape(1, 2),
      wr1, br1.reshape(1, H), wr2, br2.reshape(1, 4))


def kernel(x, edge_index, edge_type, W_conv1, Wa1, Wg1, Wb1, Wa2, Wg2, Wb2,
           Wp1, bp1, Wp2, bp2, Wr1, br1, Wr2, br2):
    src = edge_index[0]
    dst = edge_index[1]

    pad = EP - E
    srcp = jnp.pad(src, (0, pad)).reshape(EROWS, 128)
    dstp = jnp.pad(dst, (0, pad), constant_values=N).reshape(EROWS, 128)
    typep = jnp.pad(edge_type, (0, pad)).reshape(EROWS, 128)
    g2, gx, s1a, s1b = _prep(srcp, dstp, typep)

    zeros1 = jnp.zeros((ACC1_ROWS // 16, 2 * C_IN), jnp.float32)
    zeros23 = jnp.zeros((ACC23_ROWS // 16, CB), jnp.float32)

    xd = _mkxd(x)
    wbig = jnp.pad(W_conv1, ((0, 1), (0, 0), (0, 0))).reshape(C, C)
    wg1w = jnp.pad(Wg1.transpose(1, 0, 2).reshape(CB, T * CB),
                   ((0, 0), (0, 128 - T * CB)))
    wg2w = jnp.pad(Wg2.transpose(1, 0, 2).reshape(CB, T * CB),
                   ((0, 0), (0, 128 - T * CB)))
    acc1 = _sc_conv1(xd, gx, s1a, s1b, zeros1).reshape(4, N, 2 * C_IN)
    h1, ys2 = _tc1(acc1, wbig, Wa1, wg1w)

    acc2 = _sc_gconv(ys2.reshape(16 * N, CB), g2, dstp, zeros23).reshape(2, N, CB)
    h2, ys3 = _tc2(acc2, h1, Wb1, Wa2, wg2w)

    acc3 = _sc_gconv(ys3.reshape(16 * N, CB), g2, dstp, zeros23).reshape(2, N, CB)
    logits, reg = _tc3(acc3, h2, Wb2, Wp1, bp1, Wp2, bp2, Wr1, br1, Wr2, br2)
    return (logits, reg)


# speedup vs baseline: 42.3687x; 1.0479x over previous
import functools

import jax
import jax.numpy as jnp
from jax import lax
from jax.experimental import pallas as pl
from jax.experimental.pallas import tpu as pltpu
from jax.experimental.pallas import tpu_sc as plsc

N = 100000
E = 1600000
T = 7
C_IN = 4
C = 32
CB = 8
H = 32

BN = 2000
GRID = N // BN

EP = 1605632
EROWS = EP // 128
B_E = 1024
NB23 = 49
NB1 = 98
TRASH1 = 2 * N
ACC1_ROWS = 2 * N + 1088
ACC23_ROWS = N + 96
_SC_MESH = plsc.VectorSubcoreMesh(core_axis_name="c", subcore_axis_name="s")


def _prep_body(s_ref, d_ref, t_ref, g_ref, gx_ref, s1a_ref, s1b_ref):
    s = s_ref[...]
    d = d_ref[...]
    t = t_ref[...]
    g_ref[...] = 16 * s + t
    gx_ref[...] = 2 * s + (t & 1)
    valid = d < N
    tp = t >> 1
    trash = TRASH1 + (d & 1023)
    s1a_ref[...] = jnp.where((t < 4) & valid, tp * N + d, trash)
    s1b_ref[...] = jnp.where((t >= 4) & valid, (tp - 2) * N + d, trash)


def _prep(srcp, dstp, typep):
    spec = pl.BlockSpec((448, 128), lambda i: (i, 0))
    return pl.pallas_call(
        _prep_body,
        grid=(EROWS // 448,),
        in_specs=[spec, spec, spec],
        out_specs=[spec, spec, spec, spec],
        out_shape=[jax.ShapeDtypeStruct((EROWS, 128), jnp.int32)] * 4,
    )(srcp, dstp, typep)


def _gs_pipeline(nb, loadidx, table, acc, sets):
    def drain_scatters(sb, rw, ss):
        for j in range(8):
            pltpu.make_async_copy(rw.at[pl.ds(j * 128, 128)],
                                  acc.at[sb.at[j]], ss).wait()

    def do_block(b_idx, gb, sb, rw, gs, ss):
        loadidx(b_idx, gb, sb)
        descs = [pltpu.async_copy(table.at[gb.at[j]],
                                  rw.at[pl.ds(j * 128, 128)], gs)
                 for j in range(8)]
        for dsc in descs:
            dsc.wait()
        for j in range(8):
            pltpu.async_copy(rw.at[pl.ds(j * 128, 128)],
                             acc.at[sb.at[j]], ss, add=True)

    def pair(p):
        for half in (0, 1):
            gb, sb, rw, gs, ss = sets[half]
            b = 2 * p + half

            def step():
                @pl.when(p > 0)
                def _():
                    drain_scatters(sb, rw, ss)
                do_block(b, gb, sb, rw, gs, ss)

            if nb % 2 == 1 and half == 1:
                pl.when(b < nb)(step)
            else:
                step()

    pl.loop(0, (nb + 1) // 2)(pair)
    for half in (0, 1):
        gb, sb, rw, gs, ss = sets[half]
        drain_scatters(sb, rw, ss)

def _sc_conv1_body(xd_hbm, gx2_hbm, s1a_hbm, s1b_hbm, zeros_hbm, out_hbm,
                   acc, gbuf, sbuf, rows, gsem, ssem,
                   gbuf2, sbuf2, rows2, gsem2, ssem2):
    c = lax.axis_index("c")
    s = lax.axis_index("s")
    pltpu.sync_copy(zeros_hbm, acc.at[pl.ds(s * (ACC1_ROWS // 16), ACC1_ROWS // 16)])
    plsc.subcore_barrier()
    base128 = s * (NB1 * 8)

    def loadidx(b, gb, sb):
        ro = base128 + b * 8
        pltpu.sync_copy(gx2_hbm.at[pl.ds(ro, 8)], gb)

        @pl.when(c == 0)
        def _():
            pltpu.sync_copy(s1a_hbm.at[pl.ds(ro, 8)], sb)

        @pl.when(c == 1)
        def _():
            pltpu.sync_copy(s1b_hbm.at[pl.ds(ro, 8)], sb)

    _gs_pipeline(NB1, loadidx, xd_hbm, acc,
                 [(gbuf, sbuf, rows, gsem, ssem),
                  (gbuf2, sbuf2, rows2, gsem2, ssem2)])
    plsc.subcore_barrier()

    @pl.when(s < 15)
    def _():
        pltpu.sync_copy(acc.at[pl.ds(s * 12504, 12504)],
                        out_hbm.at[pl.ds(c * 2 * N + s * 12504, 12504)])

    @pl.when(s == 15)
    def _():
        pltpu.sync_copy(acc.at[pl.ds(15 * 12504, 12440)],
                        out_hbm.at[pl.ds(c * 2 * N + 15 * 12504, 12440)])


_sc_conv1 = functools.partial(
    pl.kernel,
    out_type=jax.ShapeDtypeStruct((4 * N, 2 * C_IN), jnp.float32),
    mesh=_SC_MESH,
    compiler_params=pltpu.CompilerParams(use_tc_tiling_on_sc=False),
    scratch_types=[
        pltpu.VMEM_SHARED((ACC1_ROWS, 2 * C_IN), jnp.float32),
        pltpu.VMEM((8, 128), jnp.int32),
        pltpu.VMEM((8, 128), jnp.int32),
        pltpu.VMEM((B_E, 2 * C_IN), jnp.float32),
        pltpu.SemaphoreType.DMA,
        pltpu.SemaphoreType.DMA,
        pltpu.VMEM((8, 128), jnp.int32),
        pltpu.VMEM((8, 128), jnp.int32),
        pltpu.VMEM((B_E, 2 * C_IN), jnp.float32),
        pltpu.SemaphoreType.DMA,
        pltpu.SemaphoreType.DMA,
    ],
)(_sc_conv1_body)


def _sc_gconv_body(tab_hbm, g2_hbm, d2_hbm, zeros_hbm, out_hbm,
                   acc, gbuf, sbuf, rows, gsem, ssem,
                   gbuf2, sbuf2, rows2, gsem2, ssem2):
    c = lax.axis_index("c")
    s = lax.axis_index("s")
    pltpu.sync_copy(zeros_hbm, acc.at[pl.ds(s * (ACC23_ROWS // 16), ACC23_ROWS // 16)])
    plsc.subcore_barrier()
    base128 = (c * 16 + s) * (NB23 * 8)

    def loadidx(b, gb, sb):
        ro = base128 + b * 8
        pltpu.sync_copy(g2_hbm.at[pl.ds(ro, 8)], gb)
        pltpu.sync_copy(d2_hbm.at[pl.ds(ro, 8)], sb)

    _gs_pipeline(NB23, loadidx, tab_hbm, acc,
                 [(gbuf, sbuf, rows, gsem, ssem),
                  (gbuf2, sbuf2, rows2, gsem2, ssem2)])
    plsc.subcore_barrier()
    @pl.when(s < 15)
    def _():
        pltpu.sync_copy(acc.at[pl.ds(s * 6256, 6256)],
                        out_hbm.at[pl.ds(c * N + s * 6256, 6256)])

    @pl.when(s == 15)
    def _():
        pltpu.sync_copy(acc.at[pl.ds(15 * 6256, 6160)],
                        out_hbm.at[pl.ds(c * N + 15 * 6256, 6160)])


_sc_gconv = functools.partial(
    pl.kernel,
    out_type=jax.ShapeDtypeStruct((2 * N, CB), jnp.float32),
    mesh=_SC_MESH,
    compiler_params=pltpu.CompilerParams(use_tc_tiling_on_sc=False),
    scratch_types=[
        pltpu.VMEM_SHARED((ACC23_ROWS, CB), jnp.float32),
        pltpu.VMEM((8, 128), jnp.int32),
        pltpu.VMEM((8, 128), jnp.int32),
        pltpu.VMEM((B_E, CB), jnp.float32),
        pltpu.SemaphoreType.DMA,
        pltpu.SemaphoreType.DMA,
        pltpu.VMEM((8, 128), jnp.int32),
        pltpu.VMEM((8, 128), jnp.int32),
        pltpu.VMEM((B_E, CB), jnp.float32),
        pltpu.SemaphoreType.DMA,
        pltpu.SemaphoreType.DMA,
    ],
)(_sc_gconv_body)


def _tc1_body(acc1_ref, wbig_ref, wa1_ref, wgw_ref, h1_ref, ys2_ref):
    a = acc1_ref[...]
    wbig = wbig_ref[...]
    s = jnp.zeros((BN, C), dtype=jnp.float32)
    for p in range(4):
        s = s + jnp.dot(a[p], wbig[p * 8:(p + 1) * 8],
                        preferred_element_type=jnp.float32)
    h1 = jax.nn.relu(s * (1.0 / 7.0))
    h1_ref[...] = h1
    z2 = jax.nn.relu(jnp.dot(h1, wa1_ref[...], preferred_element_type=jnp.float32))
    ys2_ref[...] = jnp.dot(z2, wgw_ref[...], preferred_element_type=jnp.float32)


def _mkxd_body(x_ref, xd_ref):
    xr = x_ref[...]
    z = jnp.zeros((BN, C_IN), jnp.float32)
    a = jnp.concatenate([xr, z], axis=1)
    b = jnp.concatenate([z, xr], axis=1)
    xd_ref[...] = jnp.stack([a, b], axis=1).reshape(2 * BN, 2 * C_IN)


def _mkxd(x):
    return pl.pallas_call(
        _mkxd_body,
        grid=(GRID,),
        in_specs=[pl.BlockSpec((BN, C_IN), lambda i: (i, 0))],
        out_specs=[pl.BlockSpec((2 * BN, 2 * C_IN), lambda i: (i, 0))],
        out_shape=[jax.ShapeDtypeStruct((2 * N, 2 * C_IN), jnp.float32)],
    )(x)[0]


def _tc1(acc1, wconv, wa1, wg1):
    return pl.pallas_call(
        _tc1_body,
        grid=(GRID,),
        in_specs=[
            pl.BlockSpec((4, BN, 2 * C_IN), lambda i: (0, i, 0)),
            pl.BlockSpec((C, C), lambda i: (0, 0)),
            pl.BlockSpec((C, CB), lambda i: (0, 0)),
            pl.BlockSpec((CB, 128), lambda i: (0, 0)),
        ],
        out_specs=[
            pl.BlockSpec((BN, C), lambda i: (i, 0)),
            pl.BlockSpec((BN, 128), lambda i: (i, 0)),
        ],
        out_shape=[
            jax.ShapeDtypeStruct((N, C), jnp.float32),
            jax.ShapeDtypeStruct((N, 128), jnp.float32),
        ],
    )(acc1, wconv, wa1, wg1)


def _tc2_body(acc_ref, h1_ref, wb_ref, wa_ref, wg_ref, h2_ref, ys3_ref):
    a = (acc_ref[0] + acc_ref[1]) * (1.0 / 7.0)
    z = jax.nn.relu(a)
    z = jnp.dot(z, wb_ref[...], preferred_element_type=jnp.float32)
    h2 = jax.nn.relu(h1_ref[...] + z)
    h2_ref[...] = h2
    z3 = jax.nn.relu(jnp.dot(h2, wa_ref[...], preferred_element_type=jnp.float32))
    ys3_ref[...] = jnp.dot(z3, wg_ref[...], preferred_element_type=jnp.float32)


def _tc2(acc2, h1, wb1, wa2, wg2):
    return pl.pallas_call(
        _tc2_body,
        grid=(GRID,),
        in_specs=[
            pl.BlockSpec((2, BN, CB), lambda i: (0, i, 0)),
            pl.BlockSpec((BN, C), lambda i: (i, 0)),
            pl.BlockSpec((CB, C), lambda i: (0, 0)),
            pl.BlockSpec((C, CB), lambda i: (0, 0)),
            pl.BlockSpec((CB, 128), lambda i: (0, 0)),
        ],
        out_specs=[
            pl.BlockSpec((BN, C), lambda i: (i, 0)),
            pl.BlockSpec((BN, 128), lambda i: (i, 0)),
        ],
        out_shape=[
            jax.ShapeDtypeStruct((N, C), jnp.float32),
            jax.ShapeDtypeStruct((N, 128), jnp.float32),
        ],
    )(acc2, h1, wb1, wa2, wg2)


def _tc3_body(acc_ref, h2_ref, wb_ref, wp1_ref, bp1_ref, wp2_ref, bp2_ref,
              wr1_ref, br1_ref, wr2_ref, br2_ref, logits_ref, reg_ref):
    a = (acc_ref[0] + acc_ref[1]) * (1.0 / 7.0)
    z = jax.nn.relu(a)
    z = jnp.dot(z, wb_ref[...], preferred_element_type=jnp.float32)
    h3 = jax.nn.relu(h2_ref[...] + z)
    p = jax.nn.relu(jnp.dot(h3, wp1_ref[...], preferred_element_type=jnp.float32)
                    + bp1_ref[...])
    logits_ref[...] = (jnp.dot(p, wp2_ref[...], preferred_element_type=jnp.float32)
                       + bp2_ref[...])
    r = jax.nn.relu(jnp.dot(h3, wr1_ref[...], preferred_element_type=jnp.float32)
                    + br1_ref[...])
    reg_ref[...] = (jnp.dot(r, wr2_ref[...], preferred_element_type=jnp.float32)
                    + br2_ref[...])


def _tc3(acc3, h2, wb2, wp1, bp1, wp2, bp2, wr1, br1, wr2, br2):
    full = lambda *s: pl.BlockSpec(s, lambda i: tuple(0 for _ in s))
    return pl.pallas_call(
        _tc3_body,
        grid=(GRID,),
        in_specs=[
            pl.BlockSpec((2, BN, CB), lambda i: (0, i, 0)),
            pl.BlockSpec((BN, C), lambda i: (i, 0)),
            full(CB, C), full(C, H), full(1, H), full(H, 2), full(1, 2),
            full(C, H), full(1, H), full(H, 4), full(1, 4),
        ],
        out_specs=[
            pl.BlockSpec((BN, 2), lambda i: (i, 0)),
            pl.BlockSpec((BN, 4), lambda i: (i, 0)),
        ],
        out_shape=[
            jax.ShapeDtypeStruct((N, 2), jnp.float32),
            jax.ShapeDtypeStruct((N, 4), jnp.float32),
        ],
    )(acc3, h2, wb2, wp1, bp1.reshape(1, H), wp2, bp2.reshape(1, 2),
      wr1, br1.reshape(1, H), wr2, br2.reshape(1, 4))


def kernel(x, edge_index, edge_type, W_conv1, Wa1, Wg1, Wb1, Wa2, Wg2, Wb2,
           Wp1, bp1, Wp2, bp2, Wr1, br1, Wr2, br2):
    src = edge_index[0]
    dst = edge_index[1]

    pad = EP - E
    srcp = jnp.pad(src, (0, pad)).reshape(EROWS, 128)
    dstp = jnp.pad(dst, (0, pad), constant_values=N).reshape(EROWS, 128)
    typep = jnp.pad(edge_type, (0, pad)).reshape(EROWS, 128)
    g2, gx, s1a, s1b = _prep(srcp, dstp, typep)

    zeros1 = jnp.zeros((ACC1_ROWS // 16, 2 * C_IN), jnp.float32)
    zeros23 = jnp.zeros((ACC23_ROWS // 16, CB), jnp.float32)

    xd = _mkxd(x)
    wbig = jnp.pad(W_conv1, ((0, 1), (0, 0), (0, 0))).reshape(C, C)
    wg1w = jnp.pad(Wg1.transpose(1, 0, 2).reshape(CB, T * CB),
                   ((0, 0), (0, 128 - T * CB)))
    wg2w = jnp.pad(Wg2.transpose(1, 0, 2).reshape(CB, T * CB),
                   ((0, 0), (0, 128 - T * CB)))
    acc1 = _sc_conv1(xd, gx, s1a, s1b, zeros1).reshape(4, N, 2 * C_IN)
    h1, ys2 = _tc1(acc1, wbig, Wa1, wg1w)

    acc2 = _sc_gconv(ys2.reshape(16 * N, CB), g2, dstp, zeros23).reshape(2, N, CB)
    h2, ys3 = _tc2(acc2, h1, Wb1, Wa2, wg2w)

    acc3 = _sc_gconv(ys3.reshape(16 * N, CB), g2, dstp, zeros23).reshape(2, N, CB)
    logits, reg = _tc3(acc3, h2, Wb2, Wp1, bp1, Wp2, bp2, Wr1, br1, Wr2, br2)
    return (logits, reg)
